# baseline (speedup 1.0000x reference)
import functools

import jax
import jax.numpy as jnp
from jax import lax
from jax.experimental import pallas as pl
from jax.experimental.pallas import tpu as pltpu
from jax.experimental.pallas import tpu_sc as plsc

B, S, CODE = 128, 32, 24
D = 256
LATENT = 128
K = 10
N = B * S
NM = 4
VOCABS = [2000, 600, 1000, 1500]
TAB_ROWS = sum(v + 1 for v in VOCABS)
OFFS = [0, 2001, 2602, 3603]

NC, NS = 2, 16
NW = NC * NS
POS_PER_W = NM * N // NW
ROWS_PER_CHUNK = 96
POS_PER_CHUNK = ROWS_PER_CHUNK // CODE
CHUNKS = POS_PER_W // POS_PER_CHUNK
ACC_POS = 128
GROUPS = POS_PER_W // ACC_POS
CHUNKS_PER_GROUP = CHUNKS // GROUPS
LANES = 16


CM = CHUNKS // NM


def _embed_body(seq_d, seq_r, seq_l, seq_p, tab_d, tab_r, tab_l, tab_p,
                out_hbm, idx_v, rows_v, acc_v, sem0, sem1):
    wid = lax.axis_index("s") * NC + lax.axis_index("c")
    seqs = (seq_d, seq_r, seq_l, seq_p)
    tabs = (tab_d, tab_r, tab_l, tab_p)
    sems = (sem0, sem1)
    for mi in range(NM):
        pltpu.sync_copy(seqs[mi].at[wid], idx_v.at[mi])

    def accum(buf, j):
        arow = j * POS_PER_CHUNK

        def pos(p, carry):
            r0 = p * CODE
            for li in range(D // LANES):
                sl = pl.ds(li * LANES, LANES)
                accv = rows_v[buf, r0, sl]
                for c in range(1, CODE):
                    accv = accv + rows_v[buf, r0 + c, sl]
                acc_v[arow + p, sl] = accv
            return carry

        lax.fori_loop(0, POS_PER_CHUNK, pos, 0)

    for mi in range(NM):
        tab = tabs[mi]

        def gather(buf, j):
            pltpu.async_copy(tab.at[idx_v.at[mi, j]], rows_v.at[buf],
                             sems[buf])

        def gwait(buf, j):
            pltpu.make_async_copy(tab.at[idx_v.at[mi, j]], rows_v.at[buf],
                                  sems[buf]).wait()

        gather(0, 0)

        def pair(jp, carry):
            j0 = 2 * jp
            j1 = j0 + 1
            gwait(0, j0)
            gather(1, j1)
            accum(0, j0)
            gwait(1, j1)

            @pl.when(jp < CM // 2 - 1)
            def _():
                gather(0, j0 + 2)

            accum(1, j1)
            return carry

        lax.fori_loop(0, CM // 2, pair, 0)
        pltpu.sync_copy(acc_v,
                        out_hbm.at[pl.ds(mi * N + wid * ACC_POS, ACC_POS)])


def _embed(seqs, tabs):
    return pl.kernel(
        _embed_body,
        out_type=jax.ShapeDtypeStruct((NM * N, D), jnp.float32),
        mesh=plsc.VectorSubcoreMesh(core_axis_name="c", subcore_axis_name="s"),
        scratch_types=[
            pltpu.VMEM((NM, CM, ROWS_PER_CHUNK), jnp.int32),
            pltpu.VMEM((2, ROWS_PER_CHUNK, D), jnp.float32),
            pltpu.VMEM((ACC_POS, D), jnp.float32),
            pltpu.SemaphoreType.DMA,
            pltpu.SemaphoreType.DMA,
        ],
    )(*seqs, *tabs)


RB = 256
NEG = float('-inf')


GF = 8
GN = N // GF


def _attend_body(hk_ref, out_ref):
    hk = hk_ref[0]
    i = pl.program_id(1)
    hq = hk_ref[0, pl.ds(i * RB, RB), :]
    sim = lax.dot_general(hq, hk, (((1,), (1,)), ((), ())),
                          preferred_element_type=jnp.float32)
    row = i * RB + lax.broadcasted_iota(jnp.int32, (RB, N), 0)
    col = lax.broadcasted_iota(jnp.int32, (RB, N), 1)
    sim = jnp.where(row == col, NEG, sim)

    x = sim.reshape(RB, GF, GN)
    m1 = jnp.max(x, axis=1)
    x2 = jnp.where(x == m1[:, None, :], NEG, x)
    m2 = jnp.max(x2, axis=1)
    x3 = jnp.where(x2 == m2[:, None, :], NEG, x2)
    m3 = jnp.max(x3, axis=1)

    y = m1
    d = jnp.zeros((RB, GN), jnp.int32)
    t = jnp.max(y, axis=1, keepdims=True)
    m = t
    for _ in range(K - 1):
        hit = y == t
        repl = jnp.where(d == 0, m2, jnp.where(d == 1, m3, NEG))
        y = jnp.where(hit, repl, y)
        d = d + hit.astype(jnp.int32)
        t = jnp.max(y, axis=1, keepdims=True)

    cnt = jnp.sum(jnp.where(sim >= t, 1.0, 0.0), axis=1, keepdims=True)
    ok = jnp.all(cnt == float(K))

    def exact(_):
        tt = jnp.max(sim, axis=1, keepdims=True)
        for _ in range(K - 1):
            tt = jnp.max(jnp.where(sim < tt, sim, NEG), axis=1, keepdims=True)
        return tt

    t = lax.cond(ok, lambda _: t, exact, 0)

    p = jnp.where(sim >= t, jnp.exp(sim - m), 0.0)
    z = jnp.sum(p, axis=1, keepdims=True)
    sel = lax.dot_general(p, hk, (((1,), (0,)), ((), ())),
                          preferred_element_type=jnp.float32)
    out_ref[0] = sel / z + hq


def _attend(h_all, interpret=False):
    return pl.pallas_call(
        _attend_body,
        grid=(NM, N // RB),
        in_specs=[
            pl.BlockSpec((1, N, D), lambda mi, i: (mi, 0, 0)),
        ],
        out_specs=pl.BlockSpec((1, RB, D), lambda mi, i: (mi, i, 0)),
        out_shape=jax.ShapeDtypeStruct((NM, N, D), jnp.float32),
        interpret=interpret,
    )(h_all)


RT = 512


def _tail_body(hbar_ref, eps_ref, Wmu_ref, bmu_ref, Wlv_ref, blv_ref,
               Wdec_ref, bdec_ref,
               W1d_ref, b1d_ref, W2d_ref, b2d_ref,
               W1r_ref, b1r_ref, W2r_ref, b2r_ref,
               W1l_ref, b1l_ref, W2l_ref, b2l_ref,
               W1p_ref, b1p_ref, W2p_ref, b2p_ref,
               od_ref, or_ref, ol_ref, op_ref, cm_ref, cl_ref):
    eps = eps_ref[:].reshape(RT, LATENT)
    Wmu = Wmu_ref[:]
    Wlv = Wlv_ref[:]
    mus, lvs = [], []
    for mi in range(NM):
        hb = hbar_ref[mi]
        mu = lax.dot_general(hb, Wmu, (((1,), (0,)), ((), ())),
                             preferred_element_type=jnp.float32) + bmu_ref[0]
        lv = lax.dot_general(hb, Wlv, (((1,), (0,)), ((), ())),
                             preferred_element_type=jnp.float32) + blv_ref[0]
        mus.append(mu)
        lvs.append(lv)
        cm_ref[:, :, mi * LATENT:(mi + 1) * LATENT] = mu.reshape(RT // S, S,
                                                                 LATENT)
        cl_ref[:, :, mi * LATENT:(mi + 1) * LATENT] = lv.reshape(RT // S, S,
                                                                 LATENT)
    mu_d, lv_d = mus[0], lvs[0]
    Wdec = Wdec_ref[:]
    W1s = (W1d_ref, W1r_ref, W1l_ref, W1p_ref)
    b1s = (b1d_ref, b1r_ref, b1l_ref, b1p_ref)
    W2s = (W2d_ref, W2r_ref, W2l_ref, W2p_ref)
    b2s = (b2d_ref, b2r_ref, b2l_ref, b2p_ref)
    outs = (od_ref, or_ref, ol_ref, op_ref)
    for mi in range(NM):
        if mi == 0:
            zm = mu_d + jnp.exp(0.5 * lv_d) * eps
        else:
            zm = (mus[mi] + mu_d) + jnp.exp(0.5 * (lvs[mi] + lv_d)) * eps
        v = lax.dot_general(zm, Wdec, (((1,), (0,)), ((), ())),
                            preferred_element_type=jnp.float32) + bdec_ref[0]
        h1 = lax.dot_general(v, W1s[mi][:], (((1,), (0,)), ((), ())),
                             preferred_element_type=jnp.float32) + b1s[mi][0]
        h1 = jnp.maximum(h1, 0.0)
        lg = lax.dot_general(h1, W2s[mi][:], (((1,), (0,)), ((), ())),
                             preferred_element_type=jnp.float32) + b2s[mi][0]
        outs[mi][:] = lg.reshape(RT // S, S, VOCABS[mi])


def _tail(hbar, eps, Wmu, bmu, Wlv, blv, Wdec, bdec, w1s, b1s, w2s, b2s,
          interpret=False):
    full = lambda shape: pl.BlockSpec(shape, lambda i: tuple(0 for _ in shape))
    in_specs = [
        pl.BlockSpec((NM, RT, D), lambda i: (0, i, 0)),
        pl.BlockSpec((RT // S, S, LATENT), lambda i: (i, 0, 0)),
        full((D, LATENT)), full((1, LATENT)),
        full((D, LATENT)), full((1, LATENT)),
        full((LATENT, D)), full((1, D)),
    ]
    args = [hbar, eps, Wmu, bmu.reshape(1, -1), Wlv, blv.reshape(1, -1),
            Wdec, bdec.reshape(1, -1)]
    for mi in range(NM):
        in_specs += [full((D, D)), full((1, D)),
                     full((D, VOCABS[mi])), full((1, VOCABS[mi]))]
        args += [w1s[mi], b1s[mi].reshape(1, -1),
                 w2s[mi], b2s[mi].reshape(1, -1)]
    out_specs = [pl.BlockSpec((RT // S, S, VOCABS[mi]), lambda i: (i, 0, 0))
                 for mi in range(NM)]
    out_specs += [pl.BlockSpec((RT // S, S, NM * LATENT),
                               lambda i: (i, 0, 0))] * 2
    out_shape = [jax.ShapeDtypeStruct((B, S, VOCABS[mi]), jnp.float32)
                 for mi in range(NM)]
    out_shape += [jax.ShapeDtypeStruct((B, S, NM * LATENT), jnp.float32)] * 2
    return pl.pallas_call(
        _tail_body,
        grid=(N // RT,),
        in_specs=in_specs,
        out_specs=out_specs,
        out_shape=out_shape,
        interpret=interpret,
    )(*args)


def kernel(diag_seq, drug_seq, lab_seq, proc_seq,
           diag_emb, drug_emb, lab_emb, proc_emb,
           diag_W1, diag_b1, diag_W2, diag_b2,
           drug_W1, drug_b1, drug_W2, drug_b2,
           lab_W1, lab_b1, lab_W2, lab_b2,
           proc_W1, proc_b1, proc_W2, proc_b2,
           W_mu, b_mu, W_lv, b_lv, W_dec, b_dec):
    seqs = [s.astype(jnp.int32).reshape(NW, CM, ROWS_PER_CHUNK)
            for s in (diag_seq, drug_seq, lab_seq, proc_seq)]
    tabs = (diag_emb, drug_emb, lab_emb, proc_emb)
    h = _embed(seqs, tabs).reshape(NM, N, D)
    hbar = _attend(h)
    eps = jax.random.normal(jax.random.key(42), (B, S, LATENT),
                            dtype=jnp.float32)
    ld, lr, ll, lp, cm, cl = _tail(
        hbar, eps, W_mu, b_mu, W_lv, b_lv, W_dec, b_dec,
        (diag_W1, drug_W1, lab_W1, proc_W1),
        (diag_b1, drug_b1, lab_b1, proc_b1),
        (diag_W2, drug_W2, lab_W2, proc_W2),
        (diag_b2, drug_b2, lab_b2, proc_b2))
    return (ld, lr, ll, lp, cm, cl)


# speedup vs baseline: 2.3200x; 2.3200x over previous
import functools

import jax
import jax.numpy as jnp
from jax import lax
from jax.experimental import pallas as pl
from jax.experimental.pallas import tpu as pltpu
from jax.experimental.pallas import tpu_sc as plsc

B, S, CODE = 128, 32, 24
D = 256
LATENT = 128
K = 10
N = B * S
NM = 4
VOCABS = [2000, 600, 1000, 1500]
TAB_ROWS = sum(v + 1 for v in VOCABS)
OFFS = [0, 2001, 2602, 3603]

NC, NS = 2, 16
NW = NC * NS
POS_PER_W = NM * N // NW
ROWS_PER_CHUNK = 96
POS_PER_CHUNK = ROWS_PER_CHUNK // CODE
CHUNKS = POS_PER_W // POS_PER_CHUNK
ACC_POS = 128
GROUPS = POS_PER_W // ACC_POS
CHUNKS_PER_GROUP = CHUNKS // GROUPS
LANES = 16


CM = CHUNKS // NM


def _embed_body(seq_d, seq_r, seq_l, seq_p, tab_d, tab_r, tab_l, tab_p,
                out_hbm, idx_v, rows_v, acc_v, sem0, sem1):
    wid = lax.axis_index("s") * NC + lax.axis_index("c")
    seqs = (seq_d, seq_r, seq_l, seq_p)
    tabs = (tab_d, tab_r, tab_l, tab_p)
    sems = (sem0, sem1)
    for mi in range(NM):
        pltpu.sync_copy(seqs[mi].at[wid], idx_v.at[mi])

    def accum(buf, j):
        arow = j * POS_PER_CHUNK

        def pos(p, carry):
            r0 = p * CODE
            for li in range(D // LANES):
                sl = pl.ds(li * LANES, LANES)
                accv = rows_v[buf, r0, sl]
                for c in range(1, CODE):
                    accv = accv + rows_v[buf, r0 + c, sl]
                acc_v[arow + p, sl] = accv
            return carry

        lax.fori_loop(0, POS_PER_CHUNK, pos, 0)

    for mi in range(NM):
        tab = tabs[mi]

        def gather(buf, j):
            pltpu.async_copy(tab.at[idx_v.at[mi, j]], rows_v.at[buf],
                             sems[buf])

        def gwait(buf, j):
            pltpu.make_async_copy(tab.at[idx_v.at[mi, j]], rows_v.at[buf],
                                  sems[buf]).wait()

        gather(0, 0)

        def pair(jp, carry):
            j0 = 2 * jp
            j1 = j0 + 1
            gwait(0, j0)
            gather(1, j1)
            accum(0, j0)
            gwait(1, j1)

            @pl.when(jp < CM // 2 - 1)
            def _():
                gather(0, j0 + 2)

            accum(1, j1)
            return carry

        lax.fori_loop(0, CM // 2, pair, 0)
        pltpu.sync_copy(acc_v,
                        out_hbm.at[pl.ds(mi * N + wid * ACC_POS, ACC_POS)])


def _embed(seqs, tabs):
    return pl.kernel(
        _embed_body,
        out_type=jax.ShapeDtypeStruct((NM * N, D), jnp.float32),
        mesh=plsc.VectorSubcoreMesh(core_axis_name="c", subcore_axis_name="s"),
        scratch_types=[
            pltpu.VMEM((NM, CM, ROWS_PER_CHUNK), jnp.int32),
            pltpu.VMEM((2, ROWS_PER_CHUNK, D), jnp.float32),
            pltpu.VMEM((ACC_POS, D), jnp.float32),
            pltpu.SemaphoreType.DMA,
            pltpu.SemaphoreType.DMA,
        ],
    )(*seqs, *tabs)


RB = 256
NEG = float('-inf')


GF = 8
GN = N // GF


def _attend_body(hk_ref, out_ref):
    hk = hk_ref[0]
    i = pl.program_id(1)
    hq = hk_ref[0, pl.ds(i * RB, RB), :]
    sim = lax.dot_general(hq, hk, (((1,), (1,)), ((), ())),
                          preferred_element_type=jnp.float32)
    row = i * RB + lax.broadcasted_iota(jnp.int32, (RB, N), 0)
    col = lax.broadcasted_iota(jnp.int32, (RB, N), 1)
    sim = jnp.where(row == col, NEG, sim)

    m1 = sim[:, 0:GN]
    m2 = jnp.full((RB, GN), NEG, jnp.float32)
    m3 = jnp.full((RB, GN), NEG, jnp.float32)
    for k in range(1, GF):
        v = sim[:, k * GN:(k + 1) * GN]
        t2 = jnp.minimum(m1, v)
        m1 = jnp.maximum(m1, v)
        t3 = jnp.minimum(m2, t2)
        m2 = jnp.maximum(m2, t2)
        m3 = jnp.maximum(m3, t3)

    y = m1
    d = jnp.zeros((RB, GN), jnp.int32)
    t = jnp.max(y, axis=1, keepdims=True)
    m = t
    for _ in range(K - 1):
        hit = y == t
        repl = jnp.where(d == 0, m2, jnp.where(d == 1, m3, NEG))
        y = jnp.where(hit, repl, y)
        d = d + hit.astype(jnp.int32)
        t = jnp.max(y, axis=1, keepdims=True)

    cnt = jnp.sum(jnp.where(sim >= t, 1.0, 0.0), axis=1, keepdims=True)
    ok = jnp.all(cnt == float(K))

    def exact(_):
        tt = jnp.max(sim, axis=1, keepdims=True)
        for _ in range(K - 1):
            tt = jnp.max(jnp.where(sim < tt, sim, NEG), axis=1, keepdims=True)
        return tt

    t = lax.cond(ok, lambda _: t, exact, 0)

    p = jnp.where(sim >= t, jnp.exp(sim - m), 0.0)
    z = jnp.sum(p, axis=1, keepdims=True)
    sel = lax.dot_general(p, hk, (((1,), (0,)), ((), ())),
                          preferred_element_type=jnp.float32)
    out_ref[0] = sel / z + hq


def _attend(h_all, interpret=False):
    return pl.pallas_call(
        _attend_body,
        grid=(NM, N // RB),
        in_specs=[
            pl.BlockSpec((1, N, D), lambda mi, i: (mi, 0, 0)),
        ],
        out_specs=pl.BlockSpec((1, RB, D), lambda mi, i: (mi, i, 0)),
        out_shape=jax.ShapeDtypeStruct((NM, N, D), jnp.float32),
        interpret=interpret,
    )(h_all)


RT = 512


def _tail_body(hbar_ref, eps_ref, Wmu_ref, bmu_ref, Wlv_ref, blv_ref,
               Wdec_ref, bdec_ref,
               W1d_ref, b1d_ref, W2d_ref, b2d_ref,
               W1r_ref, b1r_ref, W2r_ref, b2r_ref,
               W1l_ref, b1l_ref, W2l_ref, b2l_ref,
               W1p_ref, b1p_ref, W2p_ref, b2p_ref,
               od_ref, or_ref, ol_ref, op_ref, cm_ref, cl_ref):
    eps = eps_ref[:].reshape(RT, LATENT)
    Wmu = Wmu_ref[:]
    Wlv = Wlv_ref[:]
    mus, lvs = [], []
    for mi in range(NM):
        hb = hbar_ref[mi]
        mu = lax.dot_general(hb, Wmu, (((1,), (0,)), ((), ())),
                             preferred_element_type=jnp.float32) + bmu_ref[0]
        lv = lax.dot_general(hb, Wlv, (((1,), (0,)), ((), ())),
                             preferred_element_type=jnp.float32) + blv_ref[0]
        mus.append(mu)
        lvs.append(lv)
        cm_ref[:, :, mi * LATENT:(mi + 1) * LATENT] = mu.reshape(RT // S, S,
                                                                 LATENT)
        cl_ref[:, :, mi * LATENT:(mi + 1) * LATENT] = lv.reshape(RT // S, S,
                                                                 LATENT)
    mu_d, lv_d = mus[0], lvs[0]
    Wdec = Wdec_ref[:]
    W1s = (W1d_ref, W1r_ref, W1l_ref, W1p_ref)
    b1s = (b1d_ref, b1r_ref, b1l_ref, b1p_ref)
    W2s = (W2d_ref, W2r_ref, W2l_ref, W2p_ref)
    b2s = (b2d_ref, b2r_ref, b2l_ref, b2p_ref)
    outs = (od_ref, or_ref, ol_ref, op_ref)
    for mi in range(NM):
        if mi == 0:
            zm = mu_d + jnp.exp(0.5 * lv_d) * eps
        else:
            zm = (mus[mi] + mu_d) + jnp.exp(0.5 * (lvs[mi] + lv_d)) * eps
        v = lax.dot_general(zm, Wdec, (((1,), (0,)), ((), ())),
                            preferred_element_type=jnp.float32) + bdec_ref[0]
        h1 = lax.dot_general(v, W1s[mi][:], (((1,), (0,)), ((), ())),
                             preferred_element_type=jnp.float32) + b1s[mi][0]
        h1 = jnp.maximum(h1, 0.0)
        lg = lax.dot_general(h1, W2s[mi][:], (((1,), (0,)), ((), ())),
                             preferred_element_type=jnp.float32) + b2s[mi][0]
        outs[mi][:] = lg.reshape(RT // S, S, VOCABS[mi])


def _tail(hbar, eps, Wmu, bmu, Wlv, blv, Wdec, bdec, w1s, b1s, w2s, b2s,
          interpret=False):
    full = lambda shape: pl.BlockSpec(shape, lambda i: tuple(0 for _ in shape))
    in_specs = [
        pl.BlockSpec((NM, RT, D), lambda i: (0, i, 0)),
        pl.BlockSpec((RT // S, S, LATENT), lambda i: (i, 0, 0)),
        full((D, LATENT)), full((1, LATENT)),
        full((D, LATENT)), full((1, LATENT)),
        full((LATENT, D)), full((1, D)),
    ]
    args = [hbar, eps, Wmu, bmu.reshape(1, -1), Wlv, blv.reshape(1, -1),
            Wdec, bdec.reshape(1, -1)]
    for mi in range(NM):
        in_specs += [full((D, D)), full((1, D)),
                     full((D, VOCABS[mi])), full((1, VOCABS[mi]))]
        args += [w1s[mi], b1s[mi].reshape(1, -1),
                 w2s[mi], b2s[mi].reshape(1, -1)]
    out_specs = [pl.BlockSpec((RT // S, S, VOCABS[mi]), lambda i: (i, 0, 0))
                 for mi in range(NM)]
    out_specs += [pl.BlockSpec((RT // S, S, NM * LATENT),
                               lambda i: (i, 0, 0))] * 2
    out_shape = [jax.ShapeDtypeStruct((B, S, VOCABS[mi]), jnp.float32)
                 for mi in range(NM)]
    out_shape += [jax.ShapeDtypeStruct((B, S, NM * LATENT), jnp.float32)] * 2
    return pl.pallas_call(
        _tail_body,
        grid=(N // RT,),
        in_specs=in_specs,
        out_specs=out_specs,
        out_shape=out_shape,
        interpret=interpret,
    )(*args)


def kernel(diag_seq, drug_seq, lab_seq, proc_seq,
           diag_emb, drug_emb, lab_emb, proc_emb,
           diag_W1, diag_b1, diag_W2, diag_b2,
           drug_W1, drug_b1, drug_W2, drug_b2,
           lab_W1, lab_b1, lab_W2, lab_b2,
           proc_W1, proc_b1, proc_W2, proc_b2,
           W_mu, b_mu, W_lv, b_lv, W_dec, b_dec):
    seqs = [s.astype(jnp.int32).reshape(NW, CM, ROWS_PER_CHUNK)
            for s in (diag_seq, drug_seq, lab_seq, proc_seq)]
    tabs = (diag_emb, drug_emb, lab_emb, proc_emb)
    h = _embed(seqs, tabs).reshape(NM, N, D)
    hbar = _attend(h)
    eps = jax.random.normal(jax.random.key(42), (B, S, LATENT),
                            dtype=jnp.float32)
    ld, lr, ll, lp, cm, cl = _tail(
        hbar, eps, W_mu, b_mu, W_lv, b_lv, W_dec, b_dec,
        (diag_W1, drug_W1, lab_W1, proc_W1),
        (diag_b1, drug_b1, lab_b1, proc_b1),
        (diag_W2, drug_W2, lab_W2, proc_W2),
        (diag_b2, drug_b2, lab_b2, proc_b2))
    return (ld, lr, ll, lp, cm, cl)


# speedup vs baseline: 2.3335x; 1.0058x over previous
import functools

import jax
import jax.numpy as jnp
from jax import lax
from jax.experimental import pallas as pl
from jax.experimental.pallas import tpu as pltpu
from jax.experimental.pallas import tpu_sc as plsc

B, S, CODE = 128, 32, 24
D = 256
LATENT = 128
K = 10
N = B * S
NM = 4
VOCABS = [2000, 600, 1000, 1500]
TAB_ROWS = sum(v + 1 for v in VOCABS)
OFFS = [0, 2001, 2602, 3603]

NC, NS = 2, 16
NW = NC * NS
POS_PER_W = NM * N // NW
ROWS_PER_CHUNK = 96
POS_PER_CHUNK = ROWS_PER_CHUNK // CODE
CHUNKS = POS_PER_W // POS_PER_CHUNK
ACC_POS = 128
GROUPS = POS_PER_W // ACC_POS
CHUNKS_PER_GROUP = CHUNKS // GROUPS
LANES = 16


CM = CHUNKS // NM


def _embed_body(seq_d, seq_r, seq_l, seq_p, tab_d, tab_r, tab_l, tab_p,
                out_hbm, idx_v, rows_v, acc_v, sem0, sem1):
    wid = lax.axis_index("s") * NC + lax.axis_index("c")
    seqs = (seq_d, seq_r, seq_l, seq_p)
    tabs = (tab_d, tab_r, tab_l, tab_p)
    sems = (sem0, sem1)
    for mi in range(NM):
        pltpu.sync_copy(seqs[mi].at[wid], idx_v.at[mi])

    def accum(buf, j):
        arow = j * POS_PER_CHUNK

        def pos(p, carry):
            r0 = p * CODE
            for li in range(D // LANES):
                sl = pl.ds(li * LANES, LANES)
                accv = rows_v[buf, r0, sl]
                for c in range(1, CODE):
                    accv = accv + rows_v[buf, r0 + c, sl]
                acc_v[arow + p, sl] = accv
            return carry

        lax.fori_loop(0, POS_PER_CHUNK, pos, 0)

    for mi in range(NM):
        tab = tabs[mi]

        def gather(buf, j):
            pltpu.async_copy(tab.at[idx_v.at[mi, j]], rows_v.at[buf],
                             sems[buf])

        def gwait(buf, j):
            pltpu.make_async_copy(tab.at[idx_v.at[mi, j]], rows_v.at[buf],
                                  sems[buf]).wait()

        gather(0, 0)

        def pair(jp, carry):
            j0 = 2 * jp
            j1 = j0 + 1
            gwait(0, j0)
            gather(1, j1)
            accum(0, j0)
            gwait(1, j1)

            @pl.when(jp < CM // 2 - 1)
            def _():
                gather(0, j0 + 2)

            accum(1, j1)
            return carry

        lax.fori_loop(0, CM // 2, pair, 0)
        pltpu.sync_copy(acc_v,
                        out_hbm.at[pl.ds(mi * N + wid * ACC_POS, ACC_POS)])


def _embed(seqs, tabs):
    return pl.kernel(
        _embed_body,
        out_type=jax.ShapeDtypeStruct((NM * N, D), jnp.float32),
        mesh=plsc.VectorSubcoreMesh(core_axis_name="c", subcore_axis_name="s"),
        scratch_types=[
            pltpu.VMEM((NM, CM, ROWS_PER_CHUNK), jnp.int32),
            pltpu.VMEM((2, ROWS_PER_CHUNK, D), jnp.float32),
            pltpu.VMEM((ACC_POS, D), jnp.float32),
            pltpu.SemaphoreType.DMA,
            pltpu.SemaphoreType.DMA,
        ],
    )(*seqs, *tabs)


RB = 512
NEG = float('-inf')


GF = 8
GN = N // GF


def _attend_body(hk_ref, out_ref):
    hk = hk_ref[0]
    i = pl.program_id(1)
    hq = hk_ref[0, pl.ds(i * RB, RB), :]
    sim = lax.dot_general(hq, hk, (((1,), (1,)), ((), ())),
                          preferred_element_type=jnp.float32)
    row = i * RB + lax.broadcasted_iota(jnp.int32, (RB, N), 0)
    col = lax.broadcasted_iota(jnp.int32, (RB, N), 1)
    sim = jnp.where(row == col, NEG, sim)

    m1 = sim[:, 0:GN]
    m2 = jnp.full((RB, GN), NEG, jnp.float32)
    m3 = jnp.full((RB, GN), NEG, jnp.float32)
    for k in range(1, GF):
        v = sim[:, k * GN:(k + 1) * GN]
        t2 = jnp.minimum(m1, v)
        m1 = jnp.maximum(m1, v)
        t3 = jnp.minimum(m2, t2)
        m2 = jnp.maximum(m2, t2)
        m3 = jnp.maximum(m3, t3)

    y = m1
    d = jnp.zeros((RB, GN), jnp.int32)
    t = jnp.max(y, axis=1, keepdims=True)
    m = t
    for _ in range(K - 1):
        hit = y == t
        repl = jnp.where(d == 0, m2, jnp.where(d == 1, m3, NEG))
        y = jnp.where(hit, repl, y)
        d = d + hit.astype(jnp.int32)
        t = jnp.max(y, axis=1, keepdims=True)

    cnt = jnp.sum(jnp.where(sim >= t, 1.0, 0.0), axis=1, keepdims=True)
    ok = jnp.all(cnt == float(K))

    def exact(_):
        tt = jnp.max(sim, axis=1, keepdims=True)
        for _ in range(K - 1):
            tt = jnp.max(jnp.where(sim < tt, sim, NEG), axis=1, keepdims=True)
        return tt

    t = lax.cond(ok, lambda _: t, exact, 0)

    p = jnp.where(sim >= t, jnp.exp(sim - m), 0.0)
    z = jnp.sum(p, axis=1, keepdims=True)
    sel = lax.dot_general(p, hk, (((1,), (0,)), ((), ())),
                          preferred_element_type=jnp.float32)
    out_ref[0] = sel / z + hq


def _attend(h_all, interpret=False):
    return pl.pallas_call(
        _attend_body,
        grid=(NM, N // RB),
        in_specs=[
            pl.BlockSpec((1, N, D), lambda mi, i: (mi, 0, 0)),
        ],
        out_specs=pl.BlockSpec((1, RB, D), lambda mi, i: (mi, i, 0)),
        out_shape=jax.ShapeDtypeStruct((NM, N, D), jnp.float32),
        interpret=interpret,
    )(h_all)


RT = 512


def _tail_body(hbar_ref, eps_ref, Wmu_ref, bmu_ref, Wlv_ref, blv_ref,
               Wdec_ref, bdec_ref,
               W1d_ref, b1d_ref, W2d_ref, b2d_ref,
               W1r_ref, b1r_ref, W2r_ref, b2r_ref,
               W1l_ref, b1l_ref, W2l_ref, b2l_ref,
               W1p_ref, b1p_ref, W2p_ref, b2p_ref,
               od_ref, or_ref, ol_ref, op_ref, cm_ref, cl_ref):
    eps = eps_ref[:].reshape(RT, LATENT)
    Wmu = Wmu_ref[:]
    Wlv = Wlv_ref[:]
    mus, lvs = [], []
    for mi in range(NM):
        hb = hbar_ref[mi]
        mu = lax.dot_general(hb, Wmu, (((1,), (0,)), ((), ())),
                             preferred_element_type=jnp.float32) + bmu_ref[0]
        lv = lax.dot_general(hb, Wlv, (((1,), (0,)), ((), ())),
                             preferred_element_type=jnp.float32) + blv_ref[0]
        mus.append(mu)
        lvs.append(lv)
        cm_ref[:, :, mi * LATENT:(mi + 1) * LATENT] = mu.reshape(RT // S, S,
                                                                 LATENT)
        cl_ref[:, :, mi * LATENT:(mi + 1) * LATENT] = lv.reshape(RT // S, S,
                                                                 LATENT)
    mu_d, lv_d = mus[0], lvs[0]
    Wdec = Wdec_ref[:]
    W1s = (W1d_ref, W1r_ref, W1l_ref, W1p_ref)
    b1s = (b1d_ref, b1r_ref, b1l_ref, b1p_ref)
    W2s = (W2d_ref, W2r_ref, W2l_ref, W2p_ref)
    b2s = (b2d_ref, b2r_ref, b2l_ref, b2p_ref)
    outs = (od_ref, or_ref, ol_ref, op_ref)
    for mi in range(NM):
        if mi == 0:
            zm = mu_d + jnp.exp(0.5 * lv_d) * eps
        else:
            zm = (mus[mi] + mu_d) + jnp.exp(0.5 * (lvs[mi] + lv_d)) * eps
        v = lax.dot_general(zm, Wdec, (((1,), (0,)), ((), ())),
                            preferred_element_type=jnp.float32) + bdec_ref[0]
        h1 = lax.dot_general(v, W1s[mi][:], (((1,), (0,)), ((), ())),
                             preferred_element_type=jnp.float32) + b1s[mi][0]
        h1 = jnp.maximum(h1, 0.0)
        lg = lax.dot_general(h1, W2s[mi][:], (((1,), (0,)), ((), ())),
                             preferred_element_type=jnp.float32) + b2s[mi][0]
        outs[mi][:] = lg.reshape(RT // S, S, VOCABS[mi])


def _tail(hbar, eps, Wmu, bmu, Wlv, blv, Wdec, bdec, w1s, b1s, w2s, b2s,
          interpret=False):
    full = lambda shape: pl.BlockSpec(shape, lambda i: tuple(0 for _ in shape))
    in_specs = [
        pl.BlockSpec((NM, RT, D), lambda i: (0, i, 0)),
        pl.BlockSpec((RT // S, S, LATENT), lambda i: (i, 0, 0)),
        full((D, LATENT)), full((1, LATENT)),
        full((D, LATENT)), full((1, LATENT)),
        full((LATENT, D)), full((1, D)),
    ]
    args = [hbar, eps, Wmu, bmu.reshape(1, -1), Wlv, blv.reshape(1, -1),
            Wdec, bdec.reshape(1, -1)]
    for mi in range(NM):
        in_specs += [full((D, D)), full((1, D)),
                     full((D, VOCABS[mi])), full((1, VOCABS[mi]))]
        args += [w1s[mi], b1s[mi].reshape(1, -1),
                 w2s[mi], b2s[mi].reshape(1, -1)]
    out_specs = [pl.BlockSpec((RT // S, S, VOCABS[mi]), lambda i: (i, 0, 0))
                 for mi in range(NM)]
    out_specs += [pl.BlockSpec((RT // S, S, NM * LATENT),
                               lambda i: (i, 0, 0))] * 2
    out_shape = [jax.ShapeDtypeStruct((B, S, VOCABS[mi]), jnp.float32)
                 for mi in range(NM)]
    out_shape += [jax.ShapeDtypeStruct((B, S, NM * LATENT), jnp.float32)] * 2
    return pl.pallas_call(
        _tail_body,
        grid=(N // RT,),
        in_specs=in_specs,
        out_specs=out_specs,
        out_shape=out_shape,
        interpret=interpret,
    )(*args)


def kernel(diag_seq, drug_seq, lab_seq, proc_seq,
           diag_emb, drug_emb, lab_emb, proc_emb,
           diag_W1, diag_b1, diag_W2, diag_b2,
           drug_W1, drug_b1, drug_W2, drug_b2,
           lab_W1, lab_b1, lab_W2, lab_b2,
           proc_W1, proc_b1, proc_W2, proc_b2,
           W_mu, b_mu, W_lv, b_lv, W_dec, b_dec):
    seqs = [s.astype(jnp.int32).reshape(NW, CM, ROWS_PER_CHUNK)
            for s in (diag_seq, drug_seq, lab_seq, proc_seq)]
    tabs = (diag_emb, drug_emb, lab_emb, proc_emb)
    h = _embed(seqs, tabs).reshape(NM, N, D)
    hbar = _attend(h)
    eps = jax.random.normal(jax.random.key(42), (B, S, LATENT),
                            dtype=jnp.float32)
    ld, lr, ll, lp, cm, cl = _tail(
        hbar, eps, W_mu, b_mu, W_lv, b_lv, W_dec, b_dec,
        (diag_W1, drug_W1, lab_W1, proc_W1),
        (diag_b1, drug_b1, lab_b1, proc_b1),
        (diag_W2, drug_W2, lab_W2, proc_W2),
        (diag_b2, drug_b2, lab_b2, proc_b2))
    return (ld, lr, ll, lp, cm, cl)


# speedup vs baseline: 2.8573x; 1.2244x over previous
import functools

import jax
import jax.numpy as jnp
from jax import lax
from jax.experimental import pallas as pl
from jax.experimental.pallas import tpu as pltpu
from jax.experimental.pallas import tpu_sc as plsc

B, S, CODE = 128, 32, 24
D = 256
LATENT = 128
K = 10
N = B * S
NM = 4
VOCABS = [2000, 600, 1000, 1500]
TAB_ROWS = sum(v + 1 for v in VOCABS)
OFFS = [0, 2001, 2602, 3603]

NC, NS = 2, 16
NW = NC * NS
POS_PER_W = NM * N // NW
ROWS_PER_CHUNK = 96
POS_PER_CHUNK = ROWS_PER_CHUNK // CODE
CHUNKS = POS_PER_W // POS_PER_CHUNK
ACC_POS = 128
GROUPS = POS_PER_W // ACC_POS
CHUNKS_PER_GROUP = CHUNKS // GROUPS
LANES = 16


CM = CHUNKS // NM


def _make_embed_body(nm):
    def body(*args):
        seqs = args[0:nm]
        tabs = args[nm:2 * nm]
        out_hbm = args[2 * nm]
        idx_v, rows_v, acc_v, sem0, sem1 = args[2 * nm + 1:]
        _embed_impl(nm, seqs, tabs, out_hbm, idx_v, rows_v, acc_v, sem0, sem1)
    return body


def _embed_impl(nm, seqs, tabs, out_hbm, idx_v, rows_v, acc_v, sem0, sem1):
    wid = lax.axis_index("s") * NC + lax.axis_index("c")
    sems = (sem0, sem1)
    for mi in range(nm):
        pltpu.sync_copy(seqs[mi].at[wid], idx_v.at[mi])

    def accum(buf, j):
        arow = j * POS_PER_CHUNK

        def pos(p, carry):
            r0 = p * CODE
            for li in range(D // LANES):
                sl = pl.ds(li * LANES, LANES)
                accv = rows_v[buf, r0, sl]
                for c in range(1, CODE):
                    accv = accv + rows_v[buf, r0 + c, sl]
                acc_v[arow + p, sl] = accv
            return carry

        lax.fori_loop(0, POS_PER_CHUNK, pos, 0)

    for mi in range(nm):
        tab = tabs[mi]

        def gather(buf, j):
            pltpu.async_copy(tab.at[idx_v.at[mi, j]], rows_v.at[buf],
                             sems[buf])

        def gwait(buf, j):
            pltpu.make_async_copy(tab.at[idx_v.at[mi, j]], rows_v.at[buf],
                                  sems[buf]).wait()

        gather(0, 0)

        def pair(jp, carry):
            j0 = 2 * jp
            j1 = j0 + 1
            gwait(0, j0)
            gather(1, j1)
            accum(0, j0)
            gwait(1, j1)

            @pl.when(jp < CM // 2 - 1)
            def _():
                gather(0, j0 + 2)

            accum(1, j1)
            return carry

        lax.fori_loop(0, CM // 2, pair, 0)
        pltpu.sync_copy(acc_v,
                        out_hbm.at[pl.ds(mi * N + wid * ACC_POS, ACC_POS)])


def _embed(seqs, tabs):
    nm = len(seqs)
    return pl.kernel(
        _make_embed_body(nm),
        out_type=jax.ShapeDtypeStruct((nm * N, D), jnp.float32),
        mesh=plsc.VectorSubcoreMesh(core_axis_name="c", subcore_axis_name="s"),
        scratch_types=[
            pltpu.VMEM((nm, CM, ROWS_PER_CHUNK), jnp.int32),
            pltpu.VMEM((2, ROWS_PER_CHUNK, D), jnp.float32),
            pltpu.VMEM((ACC_POS, D), jnp.float32),
            pltpu.SemaphoreType.DMA,
            pltpu.SemaphoreType.DMA,
        ],
    )(*seqs, *tabs)


RB = 512
NEG = float('-inf')


GF = 8
GN = N // GF


def _attend_body(hk_ref, out_ref):
    hk = hk_ref[0]
    i = pl.program_id(1)
    hq = hk_ref[0, pl.ds(i * RB, RB), :]
    sim = lax.dot_general(hq, hk, (((1,), (1,)), ((), ())),
                          preferred_element_type=jnp.float32)
    row = i * RB + lax.broadcasted_iota(jnp.int32, (RB, N), 0)
    col = lax.broadcasted_iota(jnp.int32, (RB, N), 1)
    sim = jnp.where(row == col, NEG, sim)

    m1 = sim[:, 0:GN]
    m2 = jnp.full((RB, GN), NEG, jnp.float32)
    m3 = jnp.full((RB, GN), NEG, jnp.float32)
    for k in range(1, GF):
        v = sim[:, k * GN:(k + 1) * GN]
        t2 = jnp.minimum(m1, v)
        m1 = jnp.maximum(m1, v)
        t3 = jnp.minimum(m2, t2)
        m2 = jnp.maximum(m2, t2)
        m3 = jnp.maximum(m3, t3)

    y = m1
    d = jnp.zeros((RB, GN), jnp.int32)
    t = jnp.max(y, axis=1, keepdims=True)
    m = t
    for _ in range(K - 1):
        hit = y == t
        repl = jnp.where(d == 0, m2, jnp.where(d == 1, m3, NEG))
        y = jnp.where(hit, repl, y)
        d = d + hit.astype(jnp.int32)
        t = jnp.max(y, axis=1, keepdims=True)

    cnt = jnp.sum(jnp.where(sim >= t, 1.0, 0.0), axis=1, keepdims=True)
    ok = jnp.all(cnt == float(K))

    def exact(_):
        tt = jnp.max(sim, axis=1, keepdims=True)
        for _ in range(K - 1):
            tt = jnp.max(jnp.where(sim < tt, sim, NEG), axis=1, keepdims=True)
        return tt

    t = lax.cond(ok, lambda _: t, exact, 0)

    p = jnp.where(sim >= t, jnp.exp(sim - m), 0.0)
    z = jnp.sum(p, axis=1, keepdims=True)
    sel = lax.dot_general(p, hk, (((1,), (0,)), ((), ())),
                          preferred_element_type=jnp.float32)
    out_ref[0] = sel / z + hq


def _attend(h_all, interpret=False):
    nm = h_all.shape[0]
    return pl.pallas_call(
        _attend_body,
        grid=(nm, N // RB),
        in_specs=[
            pl.BlockSpec((1, N, D), lambda mi, i: (mi, 0, 0)),
        ],
        out_specs=pl.BlockSpec((1, RB, D), lambda mi, i: (mi, i, 0)),
        out_shape=jax.ShapeDtypeStruct((nm, N, D), jnp.float32),
        interpret=interpret,
    )(h_all)


RT = 512


def _tail_body(hbar01_ref, hbar23_ref, eps_ref, Wmu_ref, bmu_ref, Wlv_ref,
               blv_ref,
               Wdec_ref, bdec_ref,
               *rest):
    _tail_impl(hbar01_ref, hbar23_ref, eps_ref, Wmu_ref, bmu_ref, Wlv_ref,
               blv_ref, Wdec_ref, bdec_ref, *rest)


def _tail_impl(hbar01_ref, hbar23_ref, eps_ref, Wmu_ref, bmu_ref, Wlv_ref,
               blv_ref, Wdec_ref, bdec_ref,
               W1d_ref, b1d_ref, W2d_ref, b2d_ref,
               W1r_ref, b1r_ref, W2r_ref, b2r_ref,
               W1l_ref, b1l_ref, W2l_ref, b2l_ref,
               W1p_ref, b1p_ref, W2p_ref, b2p_ref,
               od_ref, or_ref, ol_ref, op_ref, cm_ref, cl_ref):
    eps = eps_ref[:].reshape(RT, LATENT)
    Wmu = Wmu_ref[:]
    Wlv = Wlv_ref[:]
    mus, lvs = [], []
    for mi in range(NM):
        hb = hbar01_ref[mi] if mi < 2 else hbar23_ref[mi - 2]
        mu = lax.dot_general(hb, Wmu, (((1,), (0,)), ((), ())),
                             preferred_element_type=jnp.float32) + bmu_ref[0]
        lv = lax.dot_general(hb, Wlv, (((1,), (0,)), ((), ())),
                             preferred_element_type=jnp.float32) + blv_ref[0]
        mus.append(mu)
        lvs.append(lv)
        cm_ref[:, :, mi * LATENT:(mi + 1) * LATENT] = mu.reshape(RT // S, S,
                                                                 LATENT)
        cl_ref[:, :, mi * LATENT:(mi + 1) * LATENT] = lv.reshape(RT // S, S,
                                                                 LATENT)
    mu_d, lv_d = mus[0], lvs[0]
    Wdec = Wdec_ref[:]
    W1s = (W1d_ref, W1r_ref, W1l_ref, W1p_ref)
    b1s = (b1d_ref, b1r_ref, b1l_ref, b1p_ref)
    W2s = (W2d_ref, W2r_ref, W2l_ref, W2p_ref)
    b2s = (b2d_ref, b2r_ref, b2l_ref, b2p_ref)
    outs = (od_ref, or_ref, ol_ref, op_ref)
    for mi in range(NM):
        if mi == 0:
            zm = mu_d + jnp.exp(0.5 * lv_d) * eps
        else:
            zm = (mus[mi] + mu_d) + jnp.exp(0.5 * (lvs[mi] + lv_d)) * eps
        v = lax.dot_general(zm, Wdec, (((1,), (0,)), ((), ())),
                            preferred_element_type=jnp.float32) + bdec_ref[0]
        h1 = lax.dot_general(v, W1s[mi][:], (((1,), (0,)), ((), ())),
                             preferred_element_type=jnp.float32) + b1s[mi][0]
        h1 = jnp.maximum(h1, 0.0)
        lg = lax.dot_general(h1, W2s[mi][:], (((1,), (0,)), ((), ())),
                             preferred_element_type=jnp.float32) + b2s[mi][0]
        outs[mi][:] = lg.reshape(RT // S, S, VOCABS[mi])


def _tail(hbar01, hbar23, eps, Wmu, bmu, Wlv, blv, Wdec, bdec,
          w1s, b1s, w2s, b2s, interpret=False):
    full = lambda shape: pl.BlockSpec(shape, lambda i: tuple(0 for _ in shape))
    in_specs = [
        pl.BlockSpec((2, RT, D), lambda i: (0, i, 0)),
        pl.BlockSpec((2, RT, D), lambda i: (0, i, 0)),
        pl.BlockSpec((RT // S, S, LATENT), lambda i: (i, 0, 0)),
        full((D, LATENT)), full((1, LATENT)),
        full((D, LATENT)), full((1, LATENT)),
        full((LATENT, D)), full((1, D)),
    ]
    args = [hbar01, hbar23, eps, Wmu, bmu.reshape(1, -1), Wlv,
            blv.reshape(1, -1), Wdec, bdec.reshape(1, -1)]
    for mi in range(NM):
        in_specs += [full((D, D)), full((1, D)),
                     full((D, VOCABS[mi])), full((1, VOCABS[mi]))]
        args += [w1s[mi], b1s[mi].reshape(1, -1),
                 w2s[mi], b2s[mi].reshape(1, -1)]
    out_specs = [pl.BlockSpec((RT // S, S, VOCABS[mi]), lambda i: (i, 0, 0))
                 for mi in range(NM)]
    out_specs += [pl.BlockSpec((RT // S, S, NM * LATENT),
                               lambda i: (i, 0, 0))] * 2
    out_shape = [jax.ShapeDtypeStruct((B, S, VOCABS[mi]), jnp.float32)
                 for mi in range(NM)]
    out_shape += [jax.ShapeDtypeStruct((B, S, NM * LATENT), jnp.float32)] * 2
    return pl.pallas_call(
        _tail_body,
        grid=(N // RT,),
        in_specs=in_specs,
        out_specs=out_specs,
        out_shape=out_shape,
        interpret=interpret,
    )(*args)


def kernel(diag_seq, drug_seq, lab_seq, proc_seq,
           diag_emb, drug_emb, lab_emb, proc_emb,
           diag_W1, diag_b1, diag_W2, diag_b2,
           drug_W1, drug_b1, drug_W2, drug_b2,
           lab_W1, lab_b1, lab_W2, lab_b2,
           proc_W1, proc_b1, proc_W2, proc_b2,
           W_mu, b_mu, W_lv, b_lv, W_dec, b_dec):
    seqs = [s.astype(jnp.int32).reshape(NW, CM, ROWS_PER_CHUNK)
            for s in (diag_seq, drug_seq, lab_seq, proc_seq)]
    h01 = _embed(seqs[0:2], (diag_emb, drug_emb)).reshape(2, N, D)
    h23 = _embed(seqs[2:4], (lab_emb, proc_emb)).reshape(2, N, D)
    hbar01 = _attend(h01)
    hbar23 = _attend(h23)
    eps = jax.random.normal(jax.random.key(42), (B, S, LATENT),
                            dtype=jnp.float32)
    ld, lr, ll, lp, cm, cl = _tail(
        hbar01, hbar23, eps, W_mu, b_mu, W_lv, b_lv, W_dec, b_dec,
        (diag_W1, drug_W1, lab_W1, proc_W1),
        (diag_b1, drug_b1, lab_b1, proc_b1),
        (diag_W2, drug_W2, lab_W2, proc_W2),
        (diag_b2, drug_b2, lab_b2, proc_b2))
    return (ld, lr, ll, lp, cm, cl)
